# Initial kernel scaffold; baseline (speedup 1.0000x reference)
import jax
import jax.numpy as jnp
from jax.experimental import pallas as pl


def kernel(delta_x_features, delta_y_i, y_innov_features, edge_index, hidden_r, pred_sigma, params):
    raise NotImplementedError("write your pallas kernel here")


# speedup vs baseline: 23.7337x; 23.7337x over previous
import functools

import jax
import jax.numpy as jnp
from jax import lax
from jax.experimental import pallas as pl
from jax.experimental.pallas import tpu as pltpu
from jax.experimental.pallas import tpu_sc as plsc

NN = 100000
NP = 100864
EE = 1600000
EP = 1605632
TPW = EP // 32
RPT = NP // 16
ZR = 788
BLK = 2048
GRID = (NN + BLK - 1) // BLK

_f32 = jnp.float32


def _sc_hist_body(col_hbm, zer_hbm, out_hbm, hist_v, idx_v, sem):
    c = lax.axis_index("c")
    s = lax.axis_index("s")
    wid = s * 2 + c

    for k in range(8):
        pltpu.sync_copy(zer_hbm, hist_v.at[pl.ds(k * (2 * RPT), 2 * RPT)])

    ones = jnp.ones((16,), _f32)
    base = wid * TPW

    def chunk(j, carry):
        pltpu.sync_copy(col_hbm.at[pl.ds(base + j * 1024, 1024)], idx_v)

        def inner(k, carry2):
            idx = idx_v[pl.ds(k * 16, 16)]
            plsc.addupdate_scatter(hist_v, [idx], ones)
            return carry2

        return lax.fori_loop(0, 64, inner, carry)

    lax.fori_loop(0, TPW // 1024, chunk, 0)
    pltpu.sync_copy(hist_v, out_hbm.at[wid])


def _sc_hist(colp, zeros1d):
    mesh = plsc.VectorSubcoreMesh(core_axis_name="c", subcore_axis_name="s")
    fn = pl.kernel(
        _sc_hist_body,
        out_type=jax.ShapeDtypeStruct((32, NP), _f32),
        mesh=mesh,
        scratch_types=[
            pltpu.VMEM((NP,), _f32),
            pltpu.VMEM((1024,), jnp.int32),
            pltpu.SemaphoreType.DMA,
        ],
        compiler_params=pltpu.CompilerParams(use_tc_tiling_on_sc=False, needs_layout_passes=False),
    )
    return fn(colp, zeros1d)


def _sc_seg_body(row_hbm, col_hbm, y_hbm, zer_hbm, out_hbm,
                 ridx_v, cidx_v, msg_v, zb_v, acc_sh, sem):
    c = lax.axis_index("c")
    s = lax.axis_index("s")
    wid = s * 2 + c
    r0 = s * RPT

    pltpu.sync_copy(zer_hbm, zb_v)
    for k in range(8):
        pltpu.sync_copy(zb_v, acc_sh.at[pl.ds(r0 + k * ZR, ZR), :])
    plsc.subcore_barrier()

    base = wid * TPW

    def chunk(j, carry):
        off = base + j * 128
        pltpu.sync_copy(row_hbm.at[pl.ds(off, 128)], ridx_v)
        pltpu.sync_copy(col_hbm.at[pl.ds(off, 128)], cidx_v.at[0])
        pltpu.async_copy(y_hbm.at[ridx_v], msg_v, sem).wait()
        pltpu.sync_copy(msg_v, acc_sh.at[cidx_v.at[0]], add=True)
        return carry

    lax.fori_loop(0, TPW // 128, chunk, 0)
    plsc.subcore_barrier()

    for k in range(8):
        pltpu.sync_copy(acc_sh.at[pl.ds(r0 + k * ZR, ZR), :], zb_v)
        pltpu.sync_copy(zb_v, out_hbm.at[c, pl.ds(r0 + k * ZR, ZR), :])


def _sc_seg(rowp, colp, y, zeros2d):
    mesh = plsc.VectorSubcoreMesh(core_axis_name="c", subcore_axis_name="s")
    fn = pl.kernel(
        _sc_seg_body,
        out_type=jax.ShapeDtypeStruct((2, NP, 16), _f32),
        mesh=mesh,
        scratch_types=[
            pltpu.VMEM((128,), jnp.int32),
            pltpu.VMEM((1, 128), jnp.int32),
            pltpu.VMEM((128, 16), _f32),
            pltpu.VMEM((ZR, 16), _f32),
            pltpu.VMEM_SHARED((NP, 16), _f32),
            pltpu.SemaphoreType.DMA,
        ],
        compiler_params=pltpu.CompilerParams(use_tc_tiling_on_sc=False, needs_layout_passes=False),
    )
    return fn(rowp, colp, y, zeros2d)


def _lrelu(v):
    return jnp.where(v > 0, v, 0.01 * v)


def _tc_a_body(yi_ref, dy_ref, hid_ref, dx_ref, ha_ref,
               dyW1, dyb1, dyW2, dyb2, rW1, rb1, rW2, rb2,
               Wih, bih, Whh, bhh, gcnW,
               sfW1, sfb1, sfW2, sfb2,
               h_out, y_out, dxf_out):
    dot = functools.partial(jnp.dot, preferred_element_type=_f32)
    yi = yi_ref[...]
    dy = dy_ref[...]
    hid = hid_ref[...]
    dx = dx_ref[...]

    a1 = _lrelu(dot(yi, dyW1[...]) + dyb1[...])
    dyi = _lrelu(dot(a1, dyW2[...]) + dyb2[...])

    r1 = _lrelu(dot(dy, rW1[...]) + rb1[...])
    rg = _lrelu(dot(r1, rW2[...]) + rb2[...])

    gi = dot(rg, Wih[...]) + bih[...]
    gh = dot(hid, Whh[...]) + bhh[...]
    r = jax.nn.sigmoid(gi[:, 0:16] + gh[:, 0:16])
    z = jax.nn.sigmoid(gi[:, 16:32] + gh[:, 16:32])
    n = jnp.tanh(gi[:, 32:48] + r * gh[:, 32:48])
    h_new = (1.0 - z) * n + z * hid
    h_out[...] = h_new

    x = dot(jnp.concatenate([dyi, h_new], axis=1), gcnW[...])
    deg = jnp.sum(ha_ref[...], axis=1, keepdims=True) + 1.0
    y_out[...] = x * lax.rsqrt(deg)

    d1 = _lrelu(dot(dx, sfW1[...]) + sfb1[...])
    dxf_out[...] = _lrelu(dot(d1, sfW2[...]) + sfb2[...])


def _tc_a(yi, dy, hid, dx, hist3, p):
    def rowspec(d):
        return pl.BlockSpec((BLK, d), lambda i: (i, 0))

    def wspec(shape):
        nd = len(shape)
        return pl.BlockSpec(shape, lambda i, nd=nd: (0,) * nd)

    ws = [p['dy_W1'].T, p['dy_b1'][None], p['dy_W2'].T, p['dy_b2'][None],
          p['r_W1'].T, p['r_b1'][None], p['r_W2'].T, p['r_b2'][None],
          p['rg_Wih'].T, p['rg_bih'][None], p['rg_Whh'].T, p['rg_bhh'][None],
          p['gcn_W'].T,
          p['sf_W1'].T, p['sf_b1'][None], p['sf_W2'].T, p['sf_b2'][None]]

    out = pl.pallas_call(
        _tc_a_body,
        grid=(GRID,),
        in_specs=[rowspec(yi.shape[1]), rowspec(8), rowspec(16), rowspec(16),
                  rowspec(32)] + [wspec(w.shape) for w in ws],
        out_specs=[rowspec(16), rowspec(16), rowspec(16)],
        out_shape=[jax.ShapeDtypeStruct((NN, 16), _f32)] * 3,
    )(yi, dy, hid, dx, hist3, *ws)
    return out


def _tc_c_body(s0_ref, s1_ref, y_ref, ha_ref, sig_ref, dxf_ref,
               gcnb, noW1, nob1, noW2, nob2, noW3, nob3,
               sgWih, sgbih, sgWhh, sgbhh,
               nko_out, ps_out):
    dot = functools.partial(jnp.dot, preferred_element_type=_f32)
    y = y_ref[...]
    deg = jnp.sum(ha_ref[...], axis=1, keepdims=True) + 1.0
    dis = lax.rsqrt(deg)
    g = dis * (s0_ref[0] + s1_ref[0] + y) / deg + gcnb[...]

    nki = jnp.concatenate([g, sig_ref[...]], axis=1)
    h1 = _lrelu(dot(nki, noW1[...]) + nob1[...])
    h2 = _lrelu(dot(h1, noW2[...]) + nob2[...])
    nko = dot(h2, noW3[...]) + nob3[...]
    nko_out[...] = nko

    gi = dot(dxf_ref[...], sgWih[...]) + sgbih[...]
    gh = dot(nko, sgWhh[...]) + sgbhh[...]
    r = jax.nn.sigmoid(gi[:, 0:8] + gh[:, 0:8])
    z = jax.nn.sigmoid(gi[:, 8:16] + gh[:, 8:16])
    n = jnp.tanh(gi[:, 16:24] + r * gh[:, 16:24])
    ps_out[...] = (1.0 - z) * n + z * nko


def _tc_c(s, y, hist3, sig, dxf, p):
    def rowspec(d):
        return pl.BlockSpec((BLK, d), lambda i: (i, 0))

    def wspec(shape):
        nd = len(shape)
        return pl.BlockSpec(shape, lambda i, nd=nd: (0,) * nd)

    sspec0 = pl.BlockSpec((1, BLK, 16), lambda i: (0, i, 0))
    sspec1 = pl.BlockSpec((1, BLK, 16), lambda i: (1, i, 0))

    ws = [p['gcn_b'][None],
          p['no_W1'].T, p['no_b1'][None], p['no_W2'].T, p['no_b2'][None],
          p['no_W3'].T, p['no_b3'][None],
          p['sg_Wih'].T, p['sg_bih'][None], p['sg_Whh'].T, p['sg_bhh'][None]]

    out = pl.pallas_call(
        _tc_c_body,
        grid=(GRID,),
        in_specs=[sspec0, sspec1, rowspec(16), rowspec(32),
                  rowspec(8), rowspec(16)] + [wspec(w.shape) for w in ws],
        out_specs=[rowspec(8), rowspec(8)],
        out_shape=[jax.ShapeDtypeStruct((NN, 8), _f32)] * 2,
    )(s, s, y, hist3, sig, dxf, *ws)
    return out


def kernel(delta_x_features, delta_y_i, y_innov_features, edge_index,
           hidden_r, pred_sigma, params):
    p = params
    row = edge_index[0].astype(jnp.int32)
    col = edge_index[1].astype(jnp.int32)
    pad = EP - EE
    rowp = jnp.concatenate([row, jnp.zeros((pad,), jnp.int32)])
    colp = jnp.concatenate([col, jnp.full((pad,), NN, jnp.int32)])
    zeros1d = jnp.zeros((2 * RPT,), _f32)
    zeros2d = jnp.zeros((ZR, 16), _f32)

    hist = _sc_hist(colp, zeros1d)
    hist3 = hist.T

    h_new, y, dxf = _tc_a(y_innov_features, delta_y_i, hidden_r[0],
                          delta_x_features, hist3, p)

    s = _sc_seg(rowp, colp, y, zeros2d)

    nko, ps_new = _tc_c(s, y, hist3, pred_sigma.astype(_f32), dxf, p)

    return (nko, h_new, h_new[None], ps_new, edge_index)


# speedup vs baseline: 29.3560x; 1.2369x over previous
import functools

import jax
import jax.numpy as jnp
from jax import lax
from jax.experimental import pallas as pl
from jax.experimental.pallas import tpu as pltpu
from jax.experimental.pallas import tpu_sc as plsc

NN = 100000
NP = 100864
EE = 1600000
EP = 1605632
TPW = EP // 32
RPT = NP // 16
ZR = 788
BLK = 2048
GRID = (NN + BLK - 1) // BLK

_f32 = jnp.float32


def _sc_hist_body(col_hbm, zer_hbm, out_hbm, hist_v, idx_v, sem):
    c = lax.axis_index("c")
    s = lax.axis_index("s")
    wid = s * 2 + c

    for k in range(8):
        pltpu.sync_copy(zer_hbm, hist_v.at[pl.ds(k * (2 * RPT), 2 * RPT)])

    ones = jnp.ones((16,), _f32)
    base = wid * TPW

    def chunk(j, carry):
        pltpu.sync_copy(col_hbm.at[pl.ds(base + j * 1024, 1024)], idx_v)

        def inner(k, carry2):
            idx = idx_v[pl.ds(k * 16, 16)]
            plsc.addupdate_scatter(hist_v, [idx], ones)
            return carry2

        return lax.fori_loop(0, 64, inner, carry)

    lax.fori_loop(0, TPW // 1024, chunk, 0)
    pltpu.sync_copy(hist_v, out_hbm.at[wid])


def _sc_hist(colp, zeros1d):
    mesh = plsc.VectorSubcoreMesh(core_axis_name="c", subcore_axis_name="s")
    fn = pl.kernel(
        _sc_hist_body,
        out_type=jax.ShapeDtypeStruct((32, NP), _f32),
        mesh=mesh,
        scratch_types=[
            pltpu.VMEM((NP,), _f32),
            pltpu.VMEM((1024,), jnp.int32),
            pltpu.SemaphoreType.DMA,
        ],
        compiler_params=pltpu.CompilerParams(use_tc_tiling_on_sc=False, needs_layout_passes=False),
    )
    return fn(colp, zeros1d)


NCH = TPW // 128


def _sc_seg_body(row_hbm, col_hbm, y_hbm, zer_hbm, out_hbm,
                 r0i, r1i, c0i, c1i, m0, m1, zb_v, acc_sh, g0, g1, si):
    c = lax.axis_index("c")
    s = lax.axis_index("s")
    wid = s * 2 + c
    r0 = s * RPT

    pltpu.sync_copy(zer_hbm, zb_v)
    for k in range(8):
        pltpu.sync_copy(zb_v, acc_sh.at[pl.ds(r0 + k * ZR, ZR), :])
    plsc.subcore_barrier()

    base = wid * NCH

    def ld_idx(j, ri, ci):
        pltpu.sync_copy(row_hbm.at[pl.ds((base + j) * 128, 128)], ri)
        pltpu.sync_copy(col_hbm.at[pl.ds(base + j, 1), :], ci)

    ld_idx(0, r0i, c0i)
    pltpu.async_copy(y_hbm.at[r0i], m0, g0)
    ld_idx(1, r1i, c1i)

    def pair(t, carry):
        j0 = t * 2
        pltpu.async_copy(y_hbm.at[r1i], m1, g1)
        pltpu.make_async_copy(y_hbm.at[r0i], m0, g0).wait()
        pltpu.sync_copy(m0, acc_sh.at[c0i.at[0]], add=True)
        j2 = jnp.minimum(j0 + 2, NCH - 2)
        ld_idx(j2, r0i, c0i)
        pltpu.async_copy(y_hbm.at[r0i], m0, g0)
        pltpu.make_async_copy(y_hbm.at[r1i], m1, g1).wait()
        pltpu.sync_copy(m1, acc_sh.at[c1i.at[0]], add=True)
        j3 = jnp.minimum(j0 + 3, NCH - 1)
        ld_idx(j3, r1i, c1i)
        return carry

    lax.fori_loop(0, NCH // 2, pair, 0)
    pltpu.make_async_copy(y_hbm.at[r0i], m0, g0).wait()
    plsc.subcore_barrier()

    for k in range(8):
        pltpu.sync_copy(acc_sh.at[pl.ds(r0 + k * ZR, ZR), :], zb_v)
        pltpu.sync_copy(zb_v, out_hbm.at[c, pl.ds(r0 + k * ZR, ZR), :])


def _sc_seg(rowp, colp2, y, zeros2d):
    mesh = plsc.VectorSubcoreMesh(core_axis_name="c", subcore_axis_name="s")
    fn = pl.kernel(
        _sc_seg_body,
        out_type=jax.ShapeDtypeStruct((2, NP, 16), _f32),
        mesh=mesh,
        scratch_types=[
            pltpu.VMEM((128,), jnp.int32),
            pltpu.VMEM((128,), jnp.int32),
            pltpu.VMEM((1, 128), jnp.int32),
            pltpu.VMEM((1, 128), jnp.int32),
            pltpu.VMEM((128, 16), _f32),
            pltpu.VMEM((128, 16), _f32),
            pltpu.VMEM((ZR, 16), _f32),
            pltpu.VMEM_SHARED((NP, 16), _f32),
            pltpu.SemaphoreType.DMA,
            pltpu.SemaphoreType.DMA,
            pltpu.SemaphoreType.DMA,
        ],
        compiler_params=pltpu.CompilerParams(use_tc_tiling_on_sc=False, needs_layout_passes=False),
    )
    return fn(rowp, colp2, y, zeros2d)


def _lrelu(v):
    return jnp.where(v > 0, v, 0.01 * v)


def _tc_a_body(yi_ref, dy_ref, hid_ref, dx_ref, ha_ref,
               dyW1, dyb1, dyW2, dyb2, rW1, rb1, rW2, rb2,
               Wih, bih, Whh, bhh, gcnW,
               sfW1, sfb1, sfW2, sfb2,
               h_out, y_out, dxf_out):
    dot = functools.partial(jnp.dot, preferred_element_type=_f32)
    yi = yi_ref[...]
    dy = dy_ref[...]
    hid = hid_ref[...]
    dx = dx_ref[...]

    a1 = _lrelu(dot(yi, dyW1[...]) + dyb1[...])
    dyi = _lrelu(dot(a1, dyW2[...]) + dyb2[...])

    r1 = _lrelu(dot(dy, rW1[...]) + rb1[...])
    rg = _lrelu(dot(r1, rW2[...]) + rb2[...])

    gi = dot(rg, Wih[...]) + bih[...]
    gh = dot(hid, Whh[...]) + bhh[...]
    r = jax.nn.sigmoid(gi[:, 0:16] + gh[:, 0:16])
    z = jax.nn.sigmoid(gi[:, 16:32] + gh[:, 16:32])
    n = jnp.tanh(gi[:, 32:48] + r * gh[:, 32:48])
    h_new = (1.0 - z) * n + z * hid
    h_out[...] = h_new

    x = dot(jnp.concatenate([dyi, h_new], axis=1), gcnW[...])
    deg = jnp.sum(ha_ref[...], axis=1, keepdims=True) + 1.0
    y_out[...] = x * lax.rsqrt(deg)

    d1 = _lrelu(dot(dx, sfW1[...]) + sfb1[...])
    dxf_out[...] = _lrelu(dot(d1, sfW2[...]) + sfb2[...])


def _tc_a(yi, dy, hid, dx, hist3, p):
    def rowspec(d):
        return pl.BlockSpec((BLK, d), lambda i: (i, 0))

    def wspec(shape):
        nd = len(shape)
        return pl.BlockSpec(shape, lambda i, nd=nd: (0,) * nd)

    ws = [p['dy_W1'].T, p['dy_b1'][None], p['dy_W2'].T, p['dy_b2'][None],
          p['r_W1'].T, p['r_b1'][None], p['r_W2'].T, p['r_b2'][None],
          p['rg_Wih'].T, p['rg_bih'][None], p['rg_Whh'].T, p['rg_bhh'][None],
          p['gcn_W'].T,
          p['sf_W1'].T, p['sf_b1'][None], p['sf_W2'].T, p['sf_b2'][None]]

    out = pl.pallas_call(
        _tc_a_body,
        grid=(GRID,),
        in_specs=[rowspec(yi.shape[1]), rowspec(8), rowspec(16), rowspec(16),
                  rowspec(32)] + [wspec(w.shape) for w in ws],
        out_specs=[rowspec(16), rowspec(16), rowspec(16)],
        out_shape=[jax.ShapeDtypeStruct((NN, 16), _f32)] * 3,
    )(yi, dy, hid, dx, hist3, *ws)
    return out


def _tc_c_body(s0_ref, s1_ref, y_ref, ha_ref, sig_ref, dxf_ref,
               gcnb, noW1, nob1, noW2, nob2, noW3, nob3,
               sgWih, sgbih, sgWhh, sgbhh,
               nko_out, ps_out):
    dot = functools.partial(jnp.dot, preferred_element_type=_f32)
    y = y_ref[...]
    deg = jnp.sum(ha_ref[...], axis=1, keepdims=True) + 1.0
    dis = lax.rsqrt(deg)
    g = dis * (s0_ref[0] + s1_ref[0] + y) / deg + gcnb[...]

    nki = jnp.concatenate([g, sig_ref[...]], axis=1)
    h1 = _lrelu(dot(nki, noW1[...]) + nob1[...])
    h2 = _lrelu(dot(h1, noW2[...]) + nob2[...])
    nko = dot(h2, noW3[...]) + nob3[...]
    nko_out[...] = nko

    gi = dot(dxf_ref[...], sgWih[...]) + sgbih[...]
    gh = dot(nko, sgWhh[...]) + sgbhh[...]
    r = jax.nn.sigmoid(gi[:, 0:8] + gh[:, 0:8])
    z = jax.nn.sigmoid(gi[:, 8:16] + gh[:, 8:16])
    n = jnp.tanh(gi[:, 16:24] + r * gh[:, 16:24])
    ps_out[...] = (1.0 - z) * n + z * nko


def _tc_c(s, y, hist3, sig, dxf, p):
    def rowspec(d):
        return pl.BlockSpec((BLK, d), lambda i: (i, 0))

    def wspec(shape):
        nd = len(shape)
        return pl.BlockSpec(shape, lambda i, nd=nd: (0,) * nd)

    sspec0 = pl.BlockSpec((1, BLK, 16), lambda i: (0, i, 0))
    sspec1 = pl.BlockSpec((1, BLK, 16), lambda i: (1, i, 0))

    ws = [p['gcn_b'][None],
          p['no_W1'].T, p['no_b1'][None], p['no_W2'].T, p['no_b2'][None],
          p['no_W3'].T, p['no_b3'][None],
          p['sg_Wih'].T, p['sg_bih'][None], p['sg_Whh'].T, p['sg_bhh'][None]]

    out = pl.pallas_call(
        _tc_c_body,
        grid=(GRID,),
        in_specs=[sspec0, sspec1, rowspec(16), rowspec(32),
                  rowspec(8), rowspec(16)] + [wspec(w.shape) for w in ws],
        out_specs=[rowspec(8), rowspec(8)],
        out_shape=[jax.ShapeDtypeStruct((NN, 8), _f32)] * 2,
    )(s, s, y, hist3, sig, dxf, *ws)
    return out


def kernel(delta_x_features, delta_y_i, y_innov_features, edge_index,
           hidden_r, pred_sigma, params):
    p = params
    row = edge_index[0].astype(jnp.int32)
    col = edge_index[1].astype(jnp.int32)
    pad = EP - EE
    rowp = jnp.concatenate([row, jnp.zeros((pad,), jnp.int32)])
    colp = jnp.concatenate([col, jnp.full((pad,), NN, jnp.int32)])
    zeros1d = jnp.zeros((2 * RPT,), _f32)
    zeros2d = jnp.zeros((ZR, 16), _f32)

    hist = _sc_hist(colp, zeros1d)
    hist3 = hist.T

    h_new, y, dxf = _tc_a(y_innov_features, delta_y_i, hidden_r[0],
                          delta_x_features, hist3, p)

    s = _sc_seg(rowp, colp.reshape(EP // 128, 128), y, zeros2d)

    nko, ps_new = _tc_c(s, y, hist3, pred_sigma.astype(_f32), dxf, p)

    return (nko, h_new, h_new[None], ps_new, edge_index)


# speedup vs baseline: 29.5618x; 1.0070x over previous
import functools

import jax
import jax.numpy as jnp
from jax import lax
from jax.experimental import pallas as pl
from jax.experimental.pallas import tpu as pltpu
from jax.experimental.pallas import tpu_sc as plsc

NN = 100000
NP = 100864
EE = 1600000
EP = 1605632
TPW = EP // 32
RPT = NP // 16
ZR = 788
BLK = 2048
GRID = (NN + BLK - 1) // BLK

_f32 = jnp.float32


def _sc_hist_body(col_hbm, zer_hbm, out_hbm, hist_v, i0, i1, h0, h1):
    c = lax.axis_index("c")
    s = lax.axis_index("s")
    wid = s * 2 + c

    for k in range(8):
        pltpu.sync_copy(zer_hbm, hist_v.at[pl.ds(k * (2 * RPT), 2 * RPT)])

    ones = jnp.ones((16,), _f32)
    base = wid * TPW
    nch = TPW // 512

    def load(j, ib, hb):
        pltpu.async_copy(col_hbm.at[pl.ds(base + j * 512, 512)], ib, hb)

    def drain(ib, hb):
        pltpu.make_async_copy(col_hbm.at[pl.ds(base, 512)], ib, hb).wait()

    def scat(ib):
        def inner(k, carry2):
            plsc.addupdate_scatter(hist_v, [ib[pl.ds(k * 16, 16)]], ones)
            return carry2
        lax.fori_loop(0, 32, inner, 0)

    load(0, i0, h0)
    load(1, i1, h1)

    def pair(t, carry):
        j0 = t * 2
        drain(i0, h0)
        scat(i0)
        load(jnp.minimum(j0 + 2, nch - 1), i0, h0)
        drain(i1, h1)
        scat(i1)
        load(jnp.minimum(j0 + 3, nch - 1), i1, h1)
        return carry

    lax.fori_loop(0, nch // 2, pair, 0)
    drain(i0, h0)
    drain(i1, h1)
    pltpu.sync_copy(hist_v, out_hbm.at[wid])


def _sc_hist(colp, zeros1d):
    mesh = plsc.VectorSubcoreMesh(core_axis_name="c", subcore_axis_name="s")
    fn = pl.kernel(
        _sc_hist_body,
        out_type=jax.ShapeDtypeStruct((32, NP), _f32),
        mesh=mesh,
        scratch_types=[
            pltpu.VMEM((NP,), _f32),
            pltpu.VMEM((512,), jnp.int32),
            pltpu.VMEM((512,), jnp.int32),
            pltpu.SemaphoreType.DMA,
            pltpu.SemaphoreType.DMA,
        ],
        compiler_params=pltpu.CompilerParams(use_tc_tiling_on_sc=False, needs_layout_passes=False),
    )
    return fn(colp, zeros1d)


NCH = TPW // 128


def _sc_seg_body(row_hbm, col_hbm, y_hbm, zer_hbm, out_hbm,
                 r0i, r1i, r2i, r3i, c0i, c1i, c2i, c3i,
                 m0, m1, m2, m3, zb_v, acc_sh, g0, g1, g2, g3):
    c = lax.axis_index("c")
    s = lax.axis_index("s")
    wid = s * 2 + c
    r0 = s * RPT

    pltpu.sync_copy(zer_hbm, zb_v)
    for k in range(8):
        pltpu.sync_copy(zb_v, acc_sh.at[pl.ds(r0 + k * ZR, ZR), :])
    plsc.subcore_barrier()

    base = wid * NCH
    slots = ((r0i, c0i, m0, g0), (r1i, c1i, m1, g1),
             (r2i, c2i, m2, g2), (r3i, c3i, m3, g3))

    def ld_idx(j, ri, ci):
        pltpu.sync_copy(row_hbm.at[pl.ds((base + j) * 128, 128)], ri)
        pltpu.sync_copy(col_hbm.at[pl.ds(base + j, 1), :], ci)

    for bslot in range(4):
        ri, ci, mb, gb = slots[bslot]
        ld_idx(bslot, ri, ci)
        pltpu.async_copy(y_hbm.at[ri], mb, gb)

    def quad(t, carry):
        j0 = t * 4
        for bslot in range(4):
            ri, ci, mb, gb = slots[bslot]
            pltpu.make_async_copy(y_hbm.at[ri], mb, gb).wait()
            pltpu.sync_copy(mb, acc_sh.at[ci.at[0]], add=True)
            jn = jnp.minimum(j0 + bslot + 4, NCH - 1)
            ld_idx(jn, ri, ci)
            pltpu.async_copy(y_hbm.at[ri], mb, gb)
        return carry

    lax.fori_loop(0, NCH // 4, quad, 0)
    for bslot in range(4):
        ri, ci, mb, gb = slots[bslot]
        pltpu.make_async_copy(y_hbm.at[ri], mb, gb).wait()
    plsc.subcore_barrier()

    for k in range(8):
        pltpu.sync_copy(acc_sh.at[pl.ds(r0 + k * ZR, ZR), :], zb_v)
        pltpu.sync_copy(zb_v, out_hbm.at[c, pl.ds(r0 + k * ZR, ZR), :])


def _sc_seg(rowp, colp2, y, zeros2d):
    mesh = plsc.VectorSubcoreMesh(core_axis_name="c", subcore_axis_name="s")
    fn = pl.kernel(
        _sc_seg_body,
        out_type=jax.ShapeDtypeStruct((2, NP, 16), _f32),
        mesh=mesh,
        scratch_types=(
            [pltpu.VMEM((128,), jnp.int32)] * 4 +
            [pltpu.VMEM((1, 128), jnp.int32)] * 4 +
            [pltpu.VMEM((128, 16), _f32)] * 4 +
            [pltpu.VMEM((ZR, 16), _f32),
             pltpu.VMEM_SHARED((NP, 16), _f32)] +
            [pltpu.SemaphoreType.DMA] * 4
        ),
        compiler_params=pltpu.CompilerParams(use_tc_tiling_on_sc=False, needs_layout_passes=False),
    )
    return fn(rowp, colp2, y, zeros2d)


def _lrelu(v):
    return jnp.where(v > 0, v, 0.01 * v)


def _tc_a_body(yi_ref, dy_ref, hid_ref, dx_ref, ha_ref,
               dyW1, dyb1, dyW2, dyb2, rW1, rb1, rW2, rb2,
               Wih, bih, Whh, bhh, gcnW,
               sfW1, sfb1, sfW2, sfb2,
               h_out, y_out, dxf_out):
    dot = functools.partial(jnp.dot, preferred_element_type=_f32)
    yi = yi_ref[...]
    dy = dy_ref[...]
    hid = hid_ref[...]
    dx = dx_ref[...]

    a1 = _lrelu(dot(yi, dyW1[...]) + dyb1[...])
    dyi = _lrelu(dot(a1, dyW2[...]) + dyb2[...])

    r1 = _lrelu(dot(dy, rW1[...]) + rb1[...])
    rg = _lrelu(dot(r1, rW2[...]) + rb2[...])

    gi = dot(rg, Wih[...]) + bih[...]
    gh = dot(hid, Whh[...]) + bhh[...]
    r = jax.nn.sigmoid(gi[:, 0:16] + gh[:, 0:16])
    z = jax.nn.sigmoid(gi[:, 16:32] + gh[:, 16:32])
    n = jnp.tanh(gi[:, 32:48] + r * gh[:, 32:48])
    h_new = (1.0 - z) * n + z * hid
    h_out[...] = h_new

    x = dot(jnp.concatenate([dyi, h_new], axis=1), gcnW[...])
    deg = jnp.sum(ha_ref[...], axis=1, keepdims=True) + 1.0
    y_out[...] = x * lax.rsqrt(deg)

    d1 = _lrelu(dot(dx, sfW1[...]) + sfb1[...])
    dxf_out[...] = _lrelu(dot(d1, sfW2[...]) + sfb2[...])


def _tc_a(yi, dy, hid, dx, hist3, p):
    def rowspec(d):
        return pl.BlockSpec((BLK, d), lambda i: (i, 0))

    def wspec(shape):
        nd = len(shape)
        return pl.BlockSpec(shape, lambda i, nd=nd: (0,) * nd)

    ws = [p['dy_W1'].T, p['dy_b1'][None], p['dy_W2'].T, p['dy_b2'][None],
          p['r_W1'].T, p['r_b1'][None], p['r_W2'].T, p['r_b2'][None],
          p['rg_Wih'].T, p['rg_bih'][None], p['rg_Whh'].T, p['rg_bhh'][None],
          p['gcn_W'].T,
          p['sf_W1'].T, p['sf_b1'][None], p['sf_W2'].T, p['sf_b2'][None]]

    out = pl.pallas_call(
        _tc_a_body,
        grid=(GRID,),
        in_specs=[rowspec(yi.shape[1]), rowspec(8), rowspec(16), rowspec(16),
                  rowspec(32)] + [wspec(w.shape) for w in ws],
        out_specs=[rowspec(16), rowspec(16), rowspec(16)],
        out_shape=[jax.ShapeDtypeStruct((NN, 16), _f32)] * 3,
    )(yi, dy, hid, dx, hist3, *ws)
    return out


def _tc_c_body(s0_ref, s1_ref, y_ref, ha_ref, sig_ref, dxf_ref,
               gcnb, noW1, nob1, noW2, nob2, noW3, nob3,
               sgWih, sgbih, sgWhh, sgbhh,
               nko_out, ps_out):
    dot = functools.partial(jnp.dot, preferred_element_type=_f32)
    y = y_ref[...]
    deg = jnp.sum(ha_ref[...], axis=1, keepdims=True) + 1.0
    dis = lax.rsqrt(deg)
    g = dis * (s0_ref[0] + s1_ref[0] + y) / deg + gcnb[...]

    nki = jnp.concatenate([g, sig_ref[...]], axis=1)
    h1 = _lrelu(dot(nki, noW1[...]) + nob1[...])
    h2 = _lrelu(dot(h1, noW2[...]) + nob2[...])
    nko = dot(h2, noW3[...]) + nob3[...]
    nko_out[...] = nko

    gi = dot(dxf_ref[...], sgWih[...]) + sgbih[...]
    gh = dot(nko, sgWhh[...]) + sgbhh[...]
    r = jax.nn.sigmoid(gi[:, 0:8] + gh[:, 0:8])
    z = jax.nn.sigmoid(gi[:, 8:16] + gh[:, 8:16])
    n = jnp.tanh(gi[:, 16:24] + r * gh[:, 16:24])
    ps_out[...] = (1.0 - z) * n + z * nko


def _tc_c(s, y, hist3, sig, dxf, p):
    def rowspec(d):
        return pl.BlockSpec((BLK, d), lambda i: (i, 0))

    def wspec(shape):
        nd = len(shape)
        return pl.BlockSpec(shape, lambda i, nd=nd: (0,) * nd)

    sspec0 = pl.BlockSpec((1, BLK, 16), lambda i: (0, i, 0))
    sspec1 = pl.BlockSpec((1, BLK, 16), lambda i: (1, i, 0))

    ws = [p['gcn_b'][None],
          p['no_W1'].T, p['no_b1'][None], p['no_W2'].T, p['no_b2'][None],
          p['no_W3'].T, p['no_b3'][None],
          p['sg_Wih'].T, p['sg_bih'][None], p['sg_Whh'].T, p['sg_bhh'][None]]

    out = pl.pallas_call(
        _tc_c_body,
        grid=(GRID,),
        in_specs=[sspec0, sspec1, rowspec(16), rowspec(32),
                  rowspec(8), rowspec(16)] + [wspec(w.shape) for w in ws],
        out_specs=[rowspec(8), rowspec(8)],
        out_shape=[jax.ShapeDtypeStruct((NN, 8), _f32)] * 2,
    )(s, s, y, hist3, sig, dxf, *ws)
    return out


def kernel(delta_x_features, delta_y_i, y_innov_features, edge_index,
           hidden_r, pred_sigma, params):
    p = params
    row = edge_index[0].astype(jnp.int32)
    col = edge_index[1].astype(jnp.int32)
    pad = EP - EE
    rowp = jnp.concatenate([row, jnp.zeros((pad,), jnp.int32)])
    colp = jnp.concatenate([col, jnp.full((pad,), NN, jnp.int32)])
    zeros1d = jnp.zeros((2 * RPT,), _f32)
    zeros2d = jnp.zeros((ZR, 16), _f32)

    hist = _sc_hist(colp, zeros1d)
    hist3 = hist.T

    h_new, y, dxf = _tc_a(y_innov_features, delta_y_i, hidden_r[0],
                          delta_x_features, hist3, p)

    s = _sc_seg(rowp, colp.reshape(EP // 128, 128), y, zeros2d)

    nko, ps_new = _tc_c(s, y, hist3, pred_sigma.astype(_f32), dxf, p)

    return (nko, h_new, h_new[None], ps_new, edge_index)


# speedup vs baseline: 36.7648x; 1.2437x over previous
import functools

import jax
import jax.numpy as jnp
from jax import lax
from jax.experimental import pallas as pl
from jax.experimental.pallas import tpu as pltpu
from jax.experimental.pallas import tpu_sc as plsc

NN = 100000
NP = 100864
EE = 1600000
EP = 1605632
TPW = EP // 32
RPT = NP // 16
ZR = 788
BLK = 2048
GRID = (NN + BLK - 1) // BLK

_f32 = jnp.float32


def _sc_hist_body(col_hbm, zer_hbm, out_hbm, hist_v, i0, i1, h0, h1):
    c = lax.axis_index("c")
    s = lax.axis_index("s")
    wid = s * 2 + c

    for k in range(8):
        pltpu.sync_copy(zer_hbm, hist_v.at[pl.ds(k * (2 * RPT), 2 * RPT)])

    ones = jnp.ones((16,), _f32)
    base = wid * TPW
    nch = TPW // 512

    def load(j, ib, hb):
        pltpu.async_copy(col_hbm.at[pl.ds(base + j * 512, 512)], ib, hb)

    def drain(ib, hb):
        pltpu.make_async_copy(col_hbm.at[pl.ds(base, 512)], ib, hb).wait()

    def scat(ib):
        def inner(k, carry2):
            plsc.addupdate_scatter(hist_v, [ib[pl.ds(k * 16, 16)]], ones)
            return carry2
        lax.fori_loop(0, 32, inner, 0)

    load(0, i0, h0)
    load(1, i1, h1)

    def pair(t, carry):
        j0 = t * 2
        drain(i0, h0)
        scat(i0)
        load(jnp.minimum(j0 + 2, nch - 1), i0, h0)
        drain(i1, h1)
        scat(i1)
        load(jnp.minimum(j0 + 3, nch - 1), i1, h1)
        return carry

    lax.fori_loop(0, nch // 2, pair, 0)
    drain(i0, h0)
    drain(i1, h1)
    pltpu.sync_copy(hist_v, out_hbm.at[wid])


def _sc_hist(colp, zeros1d):
    mesh = plsc.VectorSubcoreMesh(core_axis_name="c", subcore_axis_name="s")
    fn = pl.kernel(
        _sc_hist_body,
        out_type=jax.ShapeDtypeStruct((32, NP), _f32),
        mesh=mesh,
        scratch_types=[
            pltpu.VMEM((NP,), _f32),
            pltpu.VMEM((512,), jnp.int32),
            pltpu.VMEM((512,), jnp.int32),
            pltpu.SemaphoreType.DMA,
            pltpu.SemaphoreType.DMA,
        ],
        compiler_params=pltpu.CompilerParams(use_tc_tiling_on_sc=False, needs_layout_passes=False),
    )
    return fn(colp, zeros1d)


NCH = TPW // 128


def _sc_seg_body(row_hbm, col_hbm, y_hbm, zer_hbm, out_hbm,
                 r0i, r1i, r2i, r3i, c0i, c1i, c2i, c3i,
                 m0, m1, m2, m3, zb_v, acc_sh,
                 g0, g1, g2, g3, ir0, ir1, ir2, ir3, ic0, ic1, ic2, ic3):
    c = lax.axis_index("c")
    s = lax.axis_index("s")
    wid = s * 2 + c
    r0 = s * RPT

    pltpu.sync_copy(zer_hbm, zb_v)
    for k in range(8):
        pltpu.sync_copy(zb_v, acc_sh.at[pl.ds(r0 + k * ZR, ZR), :])
    plsc.subcore_barrier()

    base = wid * NCH
    slots = ((r0i, c0i, m0, g0, ir0, ic0), (r1i, c1i, m1, g1, ir1, ic1),
             (r2i, c2i, m2, g2, ir2, ic2), (r3i, c3i, m3, g3, ir3, ic3))

    def ld_idx(j, sl):
        pltpu.async_copy(row_hbm.at[pl.ds((base + j) * 128, 128)], sl[0], sl[4])
        pltpu.async_copy(col_hbm.at[pl.ds(base + j, 1), :], sl[1], sl[5])

    def gather(sl):
        pltpu.make_async_copy(row_hbm.at[pl.ds(base * 128, 128)], sl[0], sl[4]).wait()
        pltpu.async_copy(y_hbm.at[sl[0]], sl[2], sl[3])

    def scatter(sl):
        pltpu.make_async_copy(y_hbm.at[sl[0]], sl[2], sl[3]).wait()
        pltpu.make_async_copy(col_hbm.at[pl.ds(base, 1), :], sl[1], sl[5]).wait()
        pltpu.sync_copy(sl[2], acc_sh.at[sl[1].at[0]], add=True)

    ld_idx(0, slots[0])
    ld_idx(1, slots[1])
    gather(slots[0])
    ld_idx(2, slots[2])
    gather(slots[1])
    ld_idx(3, slots[3])

    def quad(t, carry):
        j0 = t * 4 + 2
        for v in range(4):
            sl_sc = slots[v % 4]
            sl_g = slots[(2 + v) % 4]
            scatter(sl_sc)
            ld_idx(j0 + v + 2, sl_sc)
            gather(sl_g)
        return carry

    lax.fori_loop(0, (NCH - 4) // 4, quad, 0)
    scatter(slots[0])
    gather(slots[2])
    scatter(slots[1])
    gather(slots[3])
    scatter(slots[2])
    scatter(slots[3])
    plsc.subcore_barrier()

    for k in range(8):
        pltpu.sync_copy(acc_sh.at[pl.ds(r0 + k * ZR, ZR), :], zb_v)
        pltpu.sync_copy(zb_v, out_hbm.at[c, pl.ds(r0 + k * ZR, ZR), :])


def _sc_seg(rowp, colp2, y, zeros2d):
    mesh = plsc.VectorSubcoreMesh(core_axis_name="c", subcore_axis_name="s")
    fn = pl.kernel(
        _sc_seg_body,
        out_type=jax.ShapeDtypeStruct((2, NP, 16), _f32),
        mesh=mesh,
        scratch_types=(
            [pltpu.VMEM((128,), jnp.int32)] * 4 +
            [pltpu.VMEM((1, 128), jnp.int32)] * 4 +
            [pltpu.VMEM((128, 16), _f32)] * 4 +
            [pltpu.VMEM((ZR, 16), _f32),
             pltpu.VMEM_SHARED((NP, 16), _f32)] +
            [pltpu.SemaphoreType.DMA] * 12
        ),
        compiler_params=pltpu.CompilerParams(use_tc_tiling_on_sc=False, needs_layout_passes=False),
    )
    return fn(rowp, colp2, y, zeros2d)


def _lrelu(v):
    return jnp.where(v > 0, v, 0.01 * v)


def _tc_a_body(yi_ref, dy_ref, hid_ref, dx_ref, ha_ref,
               dyW1, dyb1, dyW2, dyb2, rW1, rb1, rW2, rb2,
               Wih, bih, Whh, bhh, gcnW,
               sfW1, sfb1, sfW2, sfb2,
               h_out, y_out, dxf_out):
    dot = functools.partial(jnp.dot, preferred_element_type=_f32)
    yi = yi_ref[...]
    dy = dy_ref[...]
    hid = hid_ref[...]
    dx = dx_ref[...]

    a1 = _lrelu(dot(yi, dyW1[...]) + dyb1[...])
    dyi = _lrelu(dot(a1, dyW2[...]) + dyb2[...])

    r1 = _lrelu(dot(dy, rW1[...]) + rb1[...])
    rg = _lrelu(dot(r1, rW2[...]) + rb2[...])

    gi = dot(rg, Wih[...]) + bih[...]
    gh = dot(hid, Whh[...]) + bhh[...]
    r = jax.nn.sigmoid(gi[:, 0:16] + gh[:, 0:16])
    z = jax.nn.sigmoid(gi[:, 16:32] + gh[:, 16:32])
    n = jnp.tanh(gi[:, 32:48] + r * gh[:, 32:48])
    h_new = (1.0 - z) * n + z * hid
    h_out[...] = h_new

    x = dot(jnp.concatenate([dyi, h_new], axis=1), gcnW[...])
    deg = jnp.sum(ha_ref[...], axis=1, keepdims=True) + 1.0
    y_out[...] = x * lax.rsqrt(deg)

    d1 = _lrelu(dot(dx, sfW1[...]) + sfb1[...])
    dxf_out[...] = _lrelu(dot(d1, sfW2[...]) + sfb2[...])


def _tc_a(yi, dy, hid, dx, hist3, p):
    def rowspec(d):
        return pl.BlockSpec((BLK, d), lambda i: (i, 0))

    def wspec(shape):
        nd = len(shape)
        return pl.BlockSpec(shape, lambda i, nd=nd: (0,) * nd)

    ws = [p['dy_W1'].T, p['dy_b1'][None], p['dy_W2'].T, p['dy_b2'][None],
          p['r_W1'].T, p['r_b1'][None], p['r_W2'].T, p['r_b2'][None],
          p['rg_Wih'].T, p['rg_bih'][None], p['rg_Whh'].T, p['rg_bhh'][None],
          p['gcn_W'].T,
          p['sf_W1'].T, p['sf_b1'][None], p['sf_W2'].T, p['sf_b2'][None]]

    out = pl.pallas_call(
        _tc_a_body,
        grid=(GRID,),
        in_specs=[rowspec(yi.shape[1]), rowspec(8), rowspec(16), rowspec(16),
                  rowspec(32)] + [wspec(w.shape) for w in ws],
        out_specs=[rowspec(16), rowspec(16), rowspec(16)],
        out_shape=[jax.ShapeDtypeStruct((NN, 16), _f32)] * 3,
    )(yi, dy, hid, dx, hist3, *ws)
    return out


def _tc_c_body(s0_ref, s1_ref, y_ref, ha_ref, sig_ref, dxf_ref,
               gcnb, noW1, nob1, noW2, nob2, noW3, nob3,
               sgWih, sgbih, sgWhh, sgbhh,
               nko_out, ps_out):
    dot = functools.partial(jnp.dot, preferred_element_type=_f32)
    y = y_ref[...]
    deg = jnp.sum(ha_ref[...], axis=1, keepdims=True) + 1.0
    dis = lax.rsqrt(deg)
    g = dis * (s0_ref[0] + s1_ref[0] + y) / deg + gcnb[...]

    nki = jnp.concatenate([g, sig_ref[...]], axis=1)
    h1 = _lrelu(dot(nki, noW1[...]) + nob1[...])
    h2 = _lrelu(dot(h1, noW2[...]) + nob2[...])
    nko = dot(h2, noW3[...]) + nob3[...]
    nko_out[...] = nko

    gi = dot(dxf_ref[...], sgWih[...]) + sgbih[...]
    gh = dot(nko, sgWhh[...]) + sgbhh[...]
    r = jax.nn.sigmoid(gi[:, 0:8] + gh[:, 0:8])
    z = jax.nn.sigmoid(gi[:, 8:16] + gh[:, 8:16])
    n = jnp.tanh(gi[:, 16:24] + r * gh[:, 16:24])
    ps_out[...] = (1.0 - z) * n + z * nko


def _tc_c(s, y, hist3, sig, dxf, p):
    def rowspec(d):
        return pl.BlockSpec((BLK, d), lambda i: (i, 0))

    def wspec(shape):
        nd = len(shape)
        return pl.BlockSpec(shape, lambda i, nd=nd: (0,) * nd)

    sspec0 = pl.BlockSpec((1, BLK, 16), lambda i: (0, i, 0))
    sspec1 = pl.BlockSpec((1, BLK, 16), lambda i: (1, i, 0))

    ws = [p['gcn_b'][None],
          p['no_W1'].T, p['no_b1'][None], p['no_W2'].T, p['no_b2'][None],
          p['no_W3'].T, p['no_b3'][None],
          p['sg_Wih'].T, p['sg_bih'][None], p['sg_Whh'].T, p['sg_bhh'][None]]

    out = pl.pallas_call(
        _tc_c_body,
        grid=(GRID,),
        in_specs=[sspec0, sspec1, rowspec(16), rowspec(32),
                  rowspec(8), rowspec(16)] + [wspec(w.shape) for w in ws],
        out_specs=[rowspec(8), rowspec(8)],
        out_shape=[jax.ShapeDtypeStruct((NN, 8), _f32)] * 2,
    )(s, s, y, hist3, sig, dxf, *ws)
    return out


def kernel(delta_x_features, delta_y_i, y_innov_features, edge_index,
           hidden_r, pred_sigma, params):
    p = params
    row = edge_index[0].astype(jnp.int32)
    col = edge_index[1].astype(jnp.int32)
    pad = EP - EE
    rowp = jnp.concatenate([row, jnp.zeros((pad,), jnp.int32)])
    colp = jnp.concatenate([col, jnp.full((pad,), NN, jnp.int32)])
    zeros1d = jnp.zeros((2 * RPT,), _f32)
    zeros2d = jnp.zeros((ZR, 16), _f32)

    hist = _sc_hist(colp, zeros1d)
    hist3 = hist.T

    h_new, y, dxf = _tc_a(y_innov_features, delta_y_i, hidden_r[0],
                          delta_x_features, hist3, p)

    s = _sc_seg(rowp, colp.reshape(EP // 128, 128), y, zeros2d)

    nko, ps_new = _tc_c(s, y, hist3, pred_sigma.astype(_f32), dxf, p)

    return (nko, h_new, h_new[None], ps_new, edge_index)


# speedup vs baseline: 38.7342x; 1.0536x over previous
import functools

import jax
import jax.numpy as jnp
from jax import lax
from jax.experimental import pallas as pl
from jax.experimental.pallas import tpu as pltpu
from jax.experimental.pallas import tpu_sc as plsc

NN = 100000
NP = 100864
EE = 1600000
EP = 1605632
TPW = EP // 32
RPT = NP // 16
ZR = 788
BLK = 2048
GRID = (NN + BLK - 1) // BLK

_f32 = jnp.float32


def _sc_hist_body(col_hbm, zer_hbm, out_hbm, hist_v, i0, i1, h0, h1):
    c = lax.axis_index("c")
    s = lax.axis_index("s")
    wid = s * 2 + c

    for k in range(8):
        pltpu.sync_copy(zer_hbm, hist_v.at[pl.ds(k * (2 * RPT), 2 * RPT)])

    ones = jnp.ones((16,), _f32)
    base = wid * TPW
    nch = TPW // 512

    def load(j, ib, hb):
        pltpu.async_copy(col_hbm.at[pl.ds(base + j * 512, 512)], ib, hb)

    def drain(ib, hb):
        pltpu.make_async_copy(col_hbm.at[pl.ds(base, 512)], ib, hb).wait()

    def scat(ib):
        def inner(k, carry2):
            plsc.addupdate_scatter(hist_v, [ib[pl.ds(k * 16, 16)]], ones)
            return carry2
        lax.fori_loop(0, 32, inner, 0)

    load(0, i0, h0)
    load(1, i1, h1)

    def pair(t, carry):
        j0 = t * 2
        drain(i0, h0)
        scat(i0)
        load(jnp.minimum(j0 + 2, nch - 1), i0, h0)
        drain(i1, h1)
        scat(i1)
        load(jnp.minimum(j0 + 3, nch - 1), i1, h1)
        return carry

    lax.fori_loop(0, nch // 2, pair, 0)
    drain(i0, h0)
    drain(i1, h1)
    pltpu.sync_copy(hist_v, out_hbm.at[wid])


def _sc_hist(colp, zeros1d):
    mesh = plsc.VectorSubcoreMesh(core_axis_name="c", subcore_axis_name="s")
    fn = pl.kernel(
        _sc_hist_body,
        out_type=jax.ShapeDtypeStruct((32, NP), _f32),
        mesh=mesh,
        scratch_types=[
            pltpu.VMEM((NP,), _f32),
            pltpu.VMEM((512,), jnp.int32),
            pltpu.VMEM((512,), jnp.int32),
            pltpu.SemaphoreType.DMA,
            pltpu.SemaphoreType.DMA,
        ],
        compiler_params=pltpu.CompilerParams(use_tc_tiling_on_sc=False, needs_layout_passes=False),
    )
    return fn(colp, zeros1d)


NCH = TPW // 128


def _sc_seg_body(row_hbm, col_hbm, y_hbm, zer_hbm, out_hbm,
                 r0i, r1i, r2i, r3i, c0i, c1i, c2i, c3i,
                 m0, m1, m2, m3, zb_v, acc_sh,
                 g0, g1, g2, g3, ir0, ir1, ir2, ir3, ic0, ic1, ic2, ic3):
    c = lax.axis_index("c")
    s = lax.axis_index("s")
    wid = s * 2 + c
    r0 = s * RPT

    pltpu.sync_copy(zer_hbm, zb_v)
    for k in range(8):
        pltpu.sync_copy(zb_v, acc_sh.at[pl.ds(r0 + k * ZR, ZR), :])
    plsc.subcore_barrier()

    base = wid * NCH
    slots = ((r0i, c0i, m0, g0, ir0, ic0), (r1i, c1i, m1, g1, ir1, ic1),
             (r2i, c2i, m2, g2, ir2, ic2), (r3i, c3i, m3, g3, ir3, ic3))

    def ld_idx(j, sl):
        pltpu.async_copy(row_hbm.at[pl.ds((base + j) * 128, 128)], sl[0], sl[4])
        pltpu.async_copy(col_hbm.at[pl.ds(base + j, 1), :], sl[1], sl[5])

    def gather(sl):
        pltpu.make_async_copy(row_hbm.at[pl.ds(base * 128, 128)], sl[0], sl[4]).wait()
        pltpu.async_copy(y_hbm.at[sl[0]], sl[2], sl[3])

    def scatter(sl):
        pltpu.make_async_copy(y_hbm.at[sl[0]], sl[2], sl[3]).wait()
        pltpu.make_async_copy(col_hbm.at[pl.ds(base, 1), :], sl[1], sl[5]).wait()
        pltpu.sync_copy(sl[2], acc_sh.at[sl[1].at[0]], add=True)

    ld_idx(0, slots[0])
    ld_idx(1, slots[1])
    gather(slots[0])
    ld_idx(2, slots[2])
    gather(slots[1])
    ld_idx(3, slots[3])

    def quad(t, carry):
        j0 = t * 4 + 2
        for v in range(4):
            sl_sc = slots[v % 4]
            sl_g = slots[(2 + v) % 4]
            scatter(sl_sc)
            ld_idx(j0 + v + 2, sl_sc)
            gather(sl_g)
        return carry

    lax.fori_loop(0, (NCH - 4) // 4, quad, 0)
    scatter(slots[0])
    gather(slots[2])
    scatter(slots[1])
    gather(slots[3])
    scatter(slots[2])
    scatter(slots[3])
    plsc.subcore_barrier()

    for k in range(8):
        pltpu.sync_copy(acc_sh.at[pl.ds(r0 + k * ZR, ZR), :], zb_v)
        pltpu.sync_copy(zb_v, out_hbm.at[c, pl.ds(r0 + k * ZR, ZR), :])


def _sc_seg(rowp, colp2, y, zeros2d):
    mesh = plsc.VectorSubcoreMesh(core_axis_name="c", subcore_axis_name="s")
    fn = pl.kernel(
        _sc_seg_body,
        out_type=jax.ShapeDtypeStruct((2, NP, 16), _f32),
        mesh=mesh,
        scratch_types=(
            [pltpu.VMEM((128,), jnp.int32)] * 4 +
            [pltpu.VMEM((1, 128), jnp.int32)] * 4 +
            [pltpu.VMEM((128, 16), _f32)] * 4 +
            [pltpu.VMEM((ZR, 16), _f32),
             pltpu.VMEM_SHARED((NP, 16), _f32)] +
            [pltpu.SemaphoreType.DMA] * 12
        ),
        compiler_params=pltpu.CompilerParams(use_tc_tiling_on_sc=False, needs_layout_passes=False),
    )
    return fn(rowp, colp2, y, zeros2d)


def _lrelu(v):
    return jnp.where(v > 0, v, 0.01 * v)


def _tc_a_body(yi_ref, dy_ref, hid_ref, ha_ref,
               dyW1, dyb1, dyW2, dyb2, rW1, rb1, rW2, rb2,
               Wih, bih, Whh, bhh, gcnW,
               h_out, y_out):
    dot = functools.partial(jnp.dot, preferred_element_type=_f32)
    yi = yi_ref[...]
    dy = dy_ref[...]
    hid = hid_ref[...]

    a1 = _lrelu(dot(yi, dyW1[...]) + dyb1[...])
    dyi = _lrelu(dot(a1, dyW2[...]) + dyb2[...])

    r1 = _lrelu(dot(dy, rW1[...]) + rb1[...])
    rg = _lrelu(dot(r1, rW2[...]) + rb2[...])

    gi = dot(rg, Wih[...]) + bih[...]
    gh = dot(hid, Whh[...]) + bhh[...]
    r = jax.nn.sigmoid(gi[:, 0:16] + gh[:, 0:16])
    z = jax.nn.sigmoid(gi[:, 16:32] + gh[:, 16:32])
    n = jnp.tanh(gi[:, 32:48] + r * gh[:, 32:48])
    h_new = (1.0 - z) * n + z * hid
    h_out[...] = h_new

    x = dot(jnp.concatenate([dyi, h_new], axis=1), gcnW[...])
    deg = lax.dot_general(ha_ref[...], jnp.ones((32, 1), _f32),
                          (((0,), (0,)), ((), ())),
                          preferred_element_type=_f32) + 1.0
    y_out[...] = x * lax.rsqrt(deg)


def _tc_a(yi, dy, hid, hist, p):
    def rowspec(d):
        return pl.BlockSpec((BLK, d), lambda i: (i, 0))

    def wspec(shape):
        nd = len(shape)
        return pl.BlockSpec(shape, lambda i, nd=nd: (0,) * nd)

    hspec = pl.BlockSpec((32, BLK), lambda i: (0, i))

    ws = [p['dy_W1'].T, p['dy_b1'][None], p['dy_W2'].T, p['dy_b2'][None],
          p['r_W1'].T, p['r_b1'][None], p['r_W2'].T, p['r_b2'][None],
          p['rg_Wih'].T, p['rg_bih'][None], p['rg_Whh'].T, p['rg_bhh'][None],
          p['gcn_W'].T]

    out = pl.pallas_call(
        _tc_a_body,
        grid=(GRID,),
        in_specs=[rowspec(yi.shape[1]), rowspec(8), rowspec(16),
                  hspec] + [wspec(w.shape) for w in ws],
        out_specs=[rowspec(16), rowspec(16)],
        out_shape=[jax.ShapeDtypeStruct((NN, 16), _f32)] * 2,
    )(yi, dy, hid, hist, *ws)
    return out


def _tc_d_body(dx_ref, sfW1, sfb1, sfW2, sfb2, dxf_out):
    dot = functools.partial(jnp.dot, preferred_element_type=_f32)
    d1 = _lrelu(dot(dx_ref[...], sfW1[...]) + sfb1[...])
    dxf_out[...] = _lrelu(dot(d1, sfW2[...]) + sfb2[...])


def _tc_d(dx, p):
    def rowspec(d):
        return pl.BlockSpec((BLK, d), lambda i: (i, 0))

    def wspec(shape):
        nd = len(shape)
        return pl.BlockSpec(shape, lambda i, nd=nd: (0,) * nd)

    ws = [p['sf_W1'].T, p['sf_b1'][None], p['sf_W2'].T, p['sf_b2'][None]]
    return pl.pallas_call(
        _tc_d_body,
        grid=(GRID,),
        in_specs=[rowspec(16)] + [wspec(w.shape) for w in ws],
        out_specs=rowspec(16),
        out_shape=jax.ShapeDtypeStruct((NN, 16), _f32),
    )(dx, *ws)


def _tc_c_body(s0_ref, s1_ref, y_ref, ha_ref, sig_ref, dxf_ref,
               gcnb, noW1, nob1, noW2, nob2, noW3, nob3,
               sgWih, sgbih, sgWhh, sgbhh,
               nko_out, ps_out):
    dot = functools.partial(jnp.dot, preferred_element_type=_f32)
    y = y_ref[...]
    deg = lax.dot_general(ha_ref[...], jnp.ones((32, 1), _f32),
                          (((0,), (0,)), ((), ())),
                          preferred_element_type=_f32) + 1.0
    dis = lax.rsqrt(deg)
    g = dis * (s0_ref[0] + s1_ref[0] + y) / deg + gcnb[...]

    nki = jnp.concatenate([g, sig_ref[...]], axis=1)
    h1 = _lrelu(dot(nki, noW1[...]) + nob1[...])
    h2 = _lrelu(dot(h1, noW2[...]) + nob2[...])
    nko = dot(h2, noW3[...]) + nob3[...]
    nko_out[...] = nko

    gi = dot(dxf_ref[...], sgWih[...]) + sgbih[...]
    gh = dot(nko, sgWhh[...]) + sgbhh[...]
    r = jax.nn.sigmoid(gi[:, 0:8] + gh[:, 0:8])
    z = jax.nn.sigmoid(gi[:, 8:16] + gh[:, 8:16])
    n = jnp.tanh(gi[:, 16:24] + r * gh[:, 16:24])
    ps_out[...] = (1.0 - z) * n + z * nko


def _tc_c(s, y, hist3, sig, dxf, p):
    def rowspec(d):
        return pl.BlockSpec((BLK, d), lambda i: (i, 0))

    def wspec(shape):
        nd = len(shape)
        return pl.BlockSpec(shape, lambda i, nd=nd: (0,) * nd)

    sspec0 = pl.BlockSpec((1, BLK, 16), lambda i: (0, i, 0))
    sspec1 = pl.BlockSpec((1, BLK, 16), lambda i: (1, i, 0))

    ws = [p['gcn_b'][None],
          p['no_W1'].T, p['no_b1'][None], p['no_W2'].T, p['no_b2'][None],
          p['no_W3'].T, p['no_b3'][None],
          p['sg_Wih'].T, p['sg_bih'][None], p['sg_Whh'].T, p['sg_bhh'][None]]

    out = pl.pallas_call(
        _tc_c_body,
        grid=(GRID,),
        in_specs=[sspec0, sspec1, rowspec(16),
                  pl.BlockSpec((32, BLK), lambda i: (0, i)),
                  rowspec(8), rowspec(16)] + [wspec(w.shape) for w in ws],
        out_specs=[rowspec(8), rowspec(8)],
        out_shape=[jax.ShapeDtypeStruct((NN, 8), _f32)] * 2,
    )(s, s, y, hist3, sig, dxf, *ws)
    return out


def kernel(delta_x_features, delta_y_i, y_innov_features, edge_index,
           hidden_r, pred_sigma, params):
    p = params
    row = edge_index[0].astype(jnp.int32)
    col = edge_index[1].astype(jnp.int32)
    pad = EP - EE
    rowp = jnp.concatenate([row, jnp.zeros((pad,), jnp.int32)])
    colp = jnp.concatenate([col, jnp.full((pad,), NN, jnp.int32)])
    zeros1d = jnp.zeros((2 * RPT,), _f32)
    zeros2d = jnp.zeros((ZR, 16), _f32)

    hist = _sc_hist(colp, zeros1d)

    h_new, y = _tc_a(y_innov_features, delta_y_i, hidden_r[0], hist, p)

    s = _sc_seg(rowp, colp.reshape(EP // 128, 128), y, zeros2d)

    dxf = _tc_d(delta_x_features, p)

    nko, ps_new = _tc_c(s, y, hist, pred_sigma.astype(_f32), dxf, p)

    return (nko, h_new, h_new[None], ps_new, edge_index)
